# baseline (speedup 1.0000x reference)
import jax
import jax.numpy as jnp
import numpy as np
from jax.experimental import pallas as pl
from jax.sharding import Mesh, PartitionSpec as P


def _support_kernel(x_ref, w_ref, out_ref):
    out_ref[...] = jax.lax.dot(
        x_ref[...].astype(jnp.bfloat16),
        w_ref[...].astype(jnp.bfloat16),
        preferred_element_type=jnp.float32,
    ).astype(jnp.bfloat16)


def _spmm_kernel(adj_ref, s_ref, b_ref, out_ref):
    a = adj_ref[...].astype(jnp.bfloat16)
    acc = jax.lax.dot(a, s_ref[...], preferred_element_type=jnp.float32)
    out_ref[...] = acc + b_ref[...]


def _pick_block(n, candidates):
    for c in candidates:
        if n % c == 0:
            return c
    return n


def _impl(x, adj, W, b):
    n, d_in = x.shape
    d_out = W.shape[1]
    rows = adj.shape[0]

    bm_s = _pick_block(n, (2000, 1000, 500, 250, 8))
    support = pl.pallas_call(
        _support_kernel,
        grid=(n // bm_s,),
        in_specs=[
            pl.BlockSpec((bm_s, d_in), lambda i: (i, 0)),
            pl.BlockSpec((d_in, d_out), lambda i: (0, 0)),
        ],
        out_specs=pl.BlockSpec((bm_s, d_out), lambda i: (i, 0)),
        out_shape=jax.ShapeDtypeStruct((n, d_out), jnp.bfloat16),
    )(x, W)

    bm = _pick_block(rows, (200, 100, 50, 25, 8))
    out = pl.pallas_call(
        _spmm_kernel,
        grid=(rows // bm,),
        in_specs=[
            pl.BlockSpec((bm, n), lambda i: (i, 0)),
            pl.BlockSpec((n, d_out), lambda i: (0, 0)),
            pl.BlockSpec((1, d_out), lambda i: (0, 0)),
        ],
        out_specs=pl.BlockSpec((bm, d_out), lambda i: (i, 0)),
        out_shape=jax.ShapeDtypeStruct((rows, d_out), jnp.float32),
    )(adj, support, b)
    return out


def kernel(input, adj, W, b):
    n = adj.shape[0]
    devs = jax.devices()
    ndev = 2 if len(devs) >= 2 and n % 2 == 0 else 1
    if ndev == 1:
        return _impl(input, adj, W, b)
    mesh = Mesh(np.array(devs[:ndev]), ("d",))
    f = jax.shard_map(
        _impl,
        mesh=mesh,
        in_specs=(P(), P("d", None), P(), P()),
        out_specs=P("d", None),
        check_vma=False,
    )
    return f(input, adj, W, b)


# speedup vs baseline: 4.4820x; 4.4820x over previous
import jax
import jax.numpy as jnp
from jax.experimental import pallas as pl


def _support_kernel(x_ref, w_ref, out_ref):
    out_ref[...] = jax.lax.dot(
        x_ref[...], w_ref[...], preferred_element_type=jnp.float32
    )


def _spmm_kernel(adj_ref, s_ref, b_ref, out_ref):
    acc = jax.lax.dot(
        adj_ref[...], s_ref[...], preferred_element_type=jnp.float32
    )
    out_ref[...] = acc + b_ref[...]


def _pick_block(n, candidates):
    for c in candidates:
        if n % c == 0:
            return c
    return n


def kernel(input, adj, W, b):
    n, d_in = input.shape
    d_out = W.shape[1]

    bm_s = _pick_block(n, (2000, 1000, 500, 250, 8))
    support = pl.pallas_call(
        _support_kernel,
        grid=(n // bm_s,),
        in_specs=[
            pl.BlockSpec((bm_s, d_in), lambda i: (i, 0)),
            pl.BlockSpec((d_in, d_out), lambda i: (0, 0)),
        ],
        out_specs=pl.BlockSpec((bm_s, d_out), lambda i: (i, 0)),
        out_shape=jax.ShapeDtypeStruct((n, d_out), jnp.float32),
    )(input, W)

    bm = _pick_block(n, (400, 200, 100, 50, 25, 8))
    out = pl.pallas_call(
        _spmm_kernel,
        grid=(n // bm,),
        in_specs=[
            pl.BlockSpec((bm, n), lambda i: (i, 0)),
            pl.BlockSpec((n, d_out), lambda i: (0, 0)),
            pl.BlockSpec((1, d_out), lambda i: (0, 0)),
        ],
        out_specs=pl.BlockSpec((bm, d_out), lambda i: (i, 0)),
        out_shape=jax.ShapeDtypeStruct((n, d_out), jnp.float32),
    )(adj, support, b)
    return out


# speedup vs baseline: 4.6469x; 1.0368x over previous
import jax
import jax.numpy as jnp
from jax.experimental import pallas as pl


def _support_kernel(x_ref, w_ref, out_ref):
    out_ref[...] = jax.lax.dot(
        x_ref[...], w_ref[...], preferred_element_type=jnp.float32
    ).astype(jnp.bfloat16)


def _spmm_kernel(adj_ref, s_ref, b_ref, out_ref):
    acc = jax.lax.dot(
        adj_ref[...], s_ref[...], preferred_element_type=jnp.float32
    )
    out_ref[...] = acc + b_ref[...]


def _pick_block(n, candidates):
    for c in candidates:
        if n % c == 0:
            return c
    return n


def kernel(input, adj, W, b):
    n, d_in = input.shape
    d_out = W.shape[1]

    bm_s = _pick_block(n, (2000, 1000, 500, 250, 8))
    support = pl.pallas_call(
        _support_kernel,
        grid=(n // bm_s,),
        in_specs=[
            pl.BlockSpec((bm_s, d_in), lambda i: (i, 0)),
            pl.BlockSpec((d_in, d_out), lambda i: (0, 0)),
        ],
        out_specs=pl.BlockSpec((bm_s, d_out), lambda i: (i, 0)),
        out_shape=jax.ShapeDtypeStruct((n, d_out), jnp.bfloat16),
    )(input, W)

    bm = _pick_block(n, (400, 200, 100, 50, 25, 8))
    out = pl.pallas_call(
        _spmm_kernel,
        grid=(n // bm,),
        in_specs=[
            pl.BlockSpec((bm, n), lambda i: (i, 0)),
            pl.BlockSpec((n, d_out), lambda i: (0, 0)),
            pl.BlockSpec((1, d_out), lambda i: (0, 0)),
        ],
        out_specs=pl.BlockSpec((bm, d_out), lambda i: (i, 0)),
        out_shape=jax.ShapeDtypeStruct((n, d_out), jnp.float32),
    )(adj, support, b)
    return out
